# Initial kernel scaffold; baseline (speedup 1.0000x reference)
import jax
import jax.numpy as jnp
from jax.experimental import pallas as pl


def kernel(x, edge_index, params):
    raise NotImplementedError("write your pallas kernel here")


# speedup vs baseline: 4.4304x; 4.4304x over previous
import functools

import jax
import jax.numpy as jnp
from jax import lax
from jax.experimental import pallas as pl
from jax.experimental.pallas import tpu as pltpu
from jax.experimental.pallas import tpu_sc as plsc

_NC = 2
_NS = 16
_NW = _NC * _NS

_EPS = 1e-5


@functools.partial(jax.jit, static_argnums=(3, 4, 5))
def _sc_scatter(h, src, dst, N, D, E):
    e_per_tile = E // _NW
    CH = 80
    n_ch = e_per_tile // CH
    rows_per_tile = (N // _NS) // 8 * 8
    rows_rem = N - _NS * rows_per_tile

    mesh = plsc.VectorSubcoreMesh(core_axis_name="c", subcore_axis_name="s")

    @functools.partial(
        pl.kernel,
        out_type=jax.ShapeDtypeStruct((_NC, N, D), jnp.float32),
        mesh=mesh,
        scratch_types=[
            pltpu.VMEM((CH,), jnp.int32),
            pltpu.VMEM((CH,), jnp.int32),
            pltpu.VMEM((CH, D), jnp.float32),
            pltpu.VMEM_SHARED((N, D), jnp.float32),
            pltpu.SemaphoreType.DMA,
        ],
    )
    def scatter_kernel(h_hbm, src_hbm, dst_hbm, out_hbm,
                       src_v, dst_v, rows_v, accum, sem):
        c = lax.axis_index("c")
        s = lax.axis_index("s")
        wid = c * _NS + s

        r0 = s * rows_per_tile
        pltpu.sync_copy(h_hbm.at[pl.ds(r0, rows_per_tile)],
                        accum.at[pl.ds(r0, rows_per_tile)])
        if rows_rem:
            @pl.when(s == _NS - 1)
            def _():
                rr = _NS * rows_per_tile
                pltpu.sync_copy(h_hbm.at[pl.ds(rr, rows_rem)],
                                accum.at[pl.ds(rr, rows_rem)])
        plsc.subcore_barrier()

        base = wid * e_per_tile

        def body(j):
            off = base + j * CH
            pltpu.sync_copy(src_hbm.at[pl.ds(off, CH)], src_v)
            pltpu.sync_copy(dst_hbm.at[pl.ds(off, CH)], dst_v)
            pltpu.async_copy(h_hbm.at[src_v], rows_v, sem).wait()
            pltpu.sync_copy(rows_v, accum.at[dst_v], add=True)

        pl.loop(0, n_ch)(body)

        plsc.subcore_barrier()
        pltpu.sync_copy(accum.at[pl.ds(r0, rows_per_tile)],
                        out_hbm.at[c].at[pl.ds(r0, rows_per_tile)])
        if rows_rem:
            @pl.when(s == _NS - 1)
            def _():
                rr = _NS * rows_per_tile
                pltpu.sync_copy(accum.at[pl.ds(rr, rows_rem)],
                                out_hbm.at[c].at[pl.ds(rr, rows_rem)])

    return scatter_kernel(h, src, dst)


def _bn_relu(t, g, b):
    m = jnp.mean(t, axis=0, keepdims=True)
    v = jnp.mean((t - m) ** 2, axis=0, keepdims=True)
    return jnp.maximum(g * (t - m) * lax.rsqrt(v + _EPS) + b, 0.0)


def _dense_layer(p, h, W1, b1, g1, be1, W2, b2, gbn, bbn):
    def body(p_ref, h_ref, W1_ref, b1_ref, g1_ref, be1_ref,
             W2_ref, b2_ref, gbn_ref, bbn_ref, out_ref):
        a = p_ref[0] + p_ref[1] - h_ref[...]
        t = jnp.dot(a, W1_ref[...], preferred_element_type=jnp.float32)
        t = _bn_relu(t + b1_ref[...], g1_ref[...], be1_ref[...])
        u = jnp.dot(t, W2_ref[...], preferred_element_type=jnp.float32)
        out_ref[...] = _bn_relu(u + b2_ref[...], gbn_ref[...], bbn_ref[...])

    return pl.pallas_call(
        body,
        out_shape=jax.ShapeDtypeStruct(h.shape, jnp.float32),
    )(p, h, W1, b1, g1, be1, W2, b2, gbn, bbn)


def _dense_layer_head(p, h, W1, b1, g1, be1, W2, b2, gbn, bbn,
                      lin1_W, lin1_b, bn1_g, bn1_b, lin2_W, lin2_b, nclass):
    def body(p_ref, h_ref, W1_ref, b1_ref, g1_ref, be1_ref,
             W2_ref, b2_ref, gbn_ref, bbn_ref,
             l1W_ref, l1b_ref, g_ref, bb_ref, l2W_ref, l2b_ref, out_ref):
        a = p_ref[0] + p_ref[1] - h_ref[...]
        t = jnp.dot(a, W1_ref[...], preferred_element_type=jnp.float32)
        t = _bn_relu(t + b1_ref[...], g1_ref[...], be1_ref[...])
        u = jnp.dot(t, W2_ref[...], preferred_element_type=jnp.float32)
        hh = _bn_relu(u + b2_ref[...], gbn_ref[...], bbn_ref[...])
        t2 = jnp.dot(hh, l1W_ref[...], preferred_element_type=jnp.float32)
        t2 = _bn_relu(t2 + l1b_ref[...], g_ref[...], bb_ref[...])
        z = jnp.dot(t2, l2W_ref[...], preferred_element_type=jnp.float32)
        z = z + l2b_ref[...]
        zmax = jnp.max(z, axis=1, keepdims=True)
        ze = z - zmax
        lse = jnp.log(jnp.sum(jnp.exp(ze), axis=1, keepdims=True))
        out_ref[...] = ze - lse

    return pl.pallas_call(
        body,
        out_shape=jax.ShapeDtypeStruct((h.shape[0], nclass), jnp.float32),
    )(p, h, W1, b1, g1, be1, W2, b2, gbn, bbn,
      lin1_W, lin1_b, bn1_g, bn1_b, lin2_W, lin2_b)


def kernel(x, edge_index, params):
    N, D = x.shape
    E = edge_index.shape[1]
    src = edge_index[0]
    dst = edge_index[1]

    def row(v):
        return v.reshape(1, -1)

    h = x
    for l in range(3):
        p = _sc_scatter(h, src, dst, N, D, E)
        args = (p, h,
                params[f"W1_{l}"], row(params[f"b1_{l}"]),
                row(params[f"g1_{l}"]), row(params[f"be1_{l}"]),
                params[f"W2_{l}"], row(params[f"b2_{l}"]),
                row(params[f"gbn_{l}"]), row(params[f"bbn_{l}"]))
        if l < 2:
            h = _dense_layer(*args)
        else:
            nclass = params["lin2_W"].shape[1]
            h = _dense_layer_head(*args,
                                  params["lin1_W"], row(params["lin1_b"]),
                                  row(params["bn1_g"]), row(params["bn1_b"]),
                                  params["lin2_W"], row(params["lin2_b"]),
                                  nclass)
    return h


# speedup vs baseline: 8.4479x; 1.9068x over previous
import functools

import jax
import jax.numpy as jnp
from jax import lax
from jax.experimental import pallas as pl
from jax.experimental.pallas import tpu as pltpu
from jax.experimental.pallas import tpu_sc as plsc

_NC = 2
_NS = 16
_NW = _NC * _NS

_EPS = 1e-5


_CH = 50
_BLK = 50


@functools.partial(jax.jit, static_argnums=(3, 4, 5))
def _sc_scatter(h, src4, dst4, N, D, E):
    CH, BLK = _CH, _BLK
    e_per_tile = E // _NW
    n_ch = e_per_tile // CH
    n_blk = n_ch // BLK
    rows_per_tile = (N // _NS) // 8 * 8
    rows_rem = N - _NS * rows_per_tile

    mesh = plsc.VectorSubcoreMesh(core_axis_name="c", subcore_axis_name="s")

    @functools.partial(
        pl.kernel,
        out_type=jax.ShapeDtypeStruct((_NC, N, D), jnp.float32),
        mesh=mesh,
        scratch_types=[
            pltpu.VMEM((2, BLK, CH), jnp.int32),
            pltpu.VMEM((2, BLK, CH), jnp.int32),
            pltpu.VMEM((CH, D), jnp.float32),
            pltpu.VMEM((CH, D), jnp.float32),
            pltpu.VMEM_SHARED((N, D), jnp.float32),
            pltpu.SemaphoreType.DMA,
            pltpu.SemaphoreType.DMA,
            pltpu.SemaphoreType.DMA,
        ],
    )
    def scatter_kernel(h_hbm, src_hbm, dst_hbm, out_hbm,
                       src_blk, dst_blk, rows0, rows1, accum,
                       semi, sem0, sem1):
        c = lax.axis_index("c")
        s = lax.axis_index("s")
        wid = c * _NS + s

        def idx_start(b, par):
            pltpu.async_copy(src_hbm.at[wid].at[b], src_blk.at[par], semi)
            pltpu.async_copy(dst_hbm.at[wid].at[b], dst_blk.at[par], semi)

        def idx_wait(b, par):
            pltpu.make_async_copy(src_hbm.at[wid].at[b], src_blk.at[par],
                                  semi).wait()
            pltpu.make_async_copy(dst_hbm.at[wid].at[b], dst_blk.at[par],
                                  semi).wait()

        def gather_wait(buf, sem):
            pltpu.make_async_copy(h_hbm.at[src_blk.at[0].at[0]], buf,
                                  sem).wait()

        idx_start(0, 0)

        r0 = s * rows_per_tile
        pltpu.sync_copy(h_hbm.at[pl.ds(r0, rows_per_tile)],
                        accum.at[pl.ds(r0, rows_per_tile)])
        if rows_rem:
            @pl.when(s == _NS - 1)
            def _():
                rr = _NS * rows_per_tile
                pltpu.sync_copy(h_hbm.at[pl.ds(rr, rows_rem)],
                                accum.at[pl.ds(rr, rows_rem)])
        idx_wait(0, 0)
        plsc.subcore_barrier()

        for b in range(n_blk):
            par = b % 2
            sb = src_blk.at[par]
            db = dst_blk.at[par]
            if b + 1 < n_blk:
                idx_start(b + 1, 1 - par)

            pltpu.async_copy(h_hbm.at[sb.at[0]], rows0, sem0)

            def ibody(k, sb=sb, db=db):
                j = 2 * k
                pltpu.async_copy(h_hbm.at[sb.at[j + 1]], rows1, sem1)
                gather_wait(rows0, sem0)
                pltpu.sync_copy(rows0, accum.at[db.at[j]], add=True)

                @pl.when(j + 2 < BLK)
                def _():
                    pltpu.async_copy(h_hbm.at[sb.at[j + 2]], rows0, sem0)

                gather_wait(rows1, sem1)
                pltpu.sync_copy(rows1, accum.at[db.at[j + 1]], add=True)

            pl.loop(0, BLK // 2)(ibody)
            if b + 1 < n_blk:
                idx_wait(b + 1, 1 - par)

        plsc.subcore_barrier()
        pltpu.sync_copy(accum.at[pl.ds(r0, rows_per_tile)],
                        out_hbm.at[c].at[pl.ds(r0, rows_per_tile)])
        if rows_rem:
            @pl.when(s == _NS - 1)
            def _():
                rr = _NS * rows_per_tile
                pltpu.sync_copy(accum.at[pl.ds(rr, rows_rem)],
                                out_hbm.at[c].at[pl.ds(rr, rows_rem)])

    return scatter_kernel(h, src4, dst4)


def _bn_relu(t, g, b):
    m = jnp.mean(t, axis=0, keepdims=True)
    v = jnp.mean((t - m) ** 2, axis=0, keepdims=True)
    return jnp.maximum(g * (t - m) * lax.rsqrt(v + _EPS) + b, 0.0)


def _dense_layer(p, h, W1, b1, g1, be1, W2, b2, gbn, bbn):
    def body(p_ref, h_ref, W1_ref, b1_ref, g1_ref, be1_ref,
             W2_ref, b2_ref, gbn_ref, bbn_ref, out_ref):
        a = p_ref[0] + p_ref[1] - h_ref[...]
        t = jnp.dot(a, W1_ref[...], preferred_element_type=jnp.float32)
        t = _bn_relu(t + b1_ref[...], g1_ref[...], be1_ref[...])
        u = jnp.dot(t, W2_ref[...], preferred_element_type=jnp.float32)
        out_ref[...] = _bn_relu(u + b2_ref[...], gbn_ref[...], bbn_ref[...])

    return pl.pallas_call(
        body,
        out_shape=jax.ShapeDtypeStruct(h.shape, jnp.float32),
    )(p, h, W1, b1, g1, be1, W2, b2, gbn, bbn)


def _dense_layer_head(p, h, W1, b1, g1, be1, W2, b2, gbn, bbn,
                      lin1_W, lin1_b, bn1_g, bn1_b, lin2_W, lin2_b, nclass):
    def body(p_ref, h_ref, W1_ref, b1_ref, g1_ref, be1_ref,
             W2_ref, b2_ref, gbn_ref, bbn_ref,
             l1W_ref, l1b_ref, g_ref, bb_ref, l2W_ref, l2b_ref, out_ref):
        a = p_ref[0] + p_ref[1] - h_ref[...]
        t = jnp.dot(a, W1_ref[...], preferred_element_type=jnp.float32)
        t = _bn_relu(t + b1_ref[...], g1_ref[...], be1_ref[...])
        u = jnp.dot(t, W2_ref[...], preferred_element_type=jnp.float32)
        hh = _bn_relu(u + b2_ref[...], gbn_ref[...], bbn_ref[...])
        t2 = jnp.dot(hh, l1W_ref[...], preferred_element_type=jnp.float32)
        t2 = _bn_relu(t2 + l1b_ref[...], g_ref[...], bb_ref[...])
        z = jnp.dot(t2, l2W_ref[...], preferred_element_type=jnp.float32)
        z = z + l2b_ref[...]
        zmax = jnp.max(z, axis=1, keepdims=True)
        ze = z - zmax
        lse = jnp.log(jnp.sum(jnp.exp(ze), axis=1, keepdims=True))
        out_ref[...] = ze - lse

    return pl.pallas_call(
        body,
        out_shape=jax.ShapeDtypeStruct((h.shape[0], nclass), jnp.float32),
    )(p, h, W1, b1, g1, be1, W2, b2, gbn, bbn,
      lin1_W, lin1_b, bn1_g, bn1_b, lin2_W, lin2_b)


def kernel(x, edge_index, params):
    N, D = x.shape
    E = edge_index.shape[1]
    n_blk = E // _NW // _CH // _BLK
    src = edge_index[0].reshape(_NW, n_blk, _BLK, _CH)
    dst = edge_index[1].reshape(_NW, n_blk, _BLK, _CH)

    def row(v):
        return v.reshape(1, -1)

    h = x
    for l in range(3):
        p = _sc_scatter(h, src, dst, N, D, E)
        args = (p, h,
                params[f"W1_{l}"], row(params[f"b1_{l}"]),
                row(params[f"g1_{l}"]), row(params[f"be1_{l}"]),
                params[f"W2_{l}"], row(params[f"b2_{l}"]),
                row(params[f"gbn_{l}"]), row(params[f"bbn_{l}"]))
        if l < 2:
            h = _dense_layer(*args)
        else:
            nclass = params["lin2_W"].shape[1]
            h = _dense_layer_head(*args,
                                  params["lin1_W"], row(params["lin1_b"]),
                                  row(params["bn1_g"]), row(params["bn1_b"]),
                                  params["lin2_W"], row(params["lin2_b"]),
                                  nclass)
    return h
